# baseline (speedup 1.0000x reference)
import jax
import jax.numpy as jnp
from jax.experimental import pallas as pl
from jax.experimental.pallas import tpu as pltpu

N, NA, NS = 1024, 4096, 60
F0, F1, F2 = 1902, 334, 64
H, D = 256, 64
TAU = 0.5

EG = 4
PG = 4


def _elu(x):
    return jnp.where(x > 0, x, jnp.exp(x) - 1.0)


def _normalize(x):
    nrm = jnp.sqrt(jnp.sum(x * x, axis=1, keepdims=True))
    return x / jnp.clip(nrm, 1e-12)


def _dot(a, b):
    return jnp.dot(a, b, preferred_element_type=jnp.float32)


def _encode_kernel(feat0_ref, mask_ref, w0_ref, b0_ref, htar_ref, hmask_ref):
    w0 = w0_ref[...]
    b0 = b0_ref[...]
    htar_ref[...] = _elu(_dot(feat0_ref[...], w0) + b0)
    hmask_ref[...] = _elu(_dot(mask_ref[...], w0) + b0)


def _agg_kernel(feat1_ref, w1_ref, b1_ref, nei0_ref, nei1_ref, feat2_ref,
                w2_ref, b2_ref, htar_ref, hmask_ref, wagg0_ref, wagg1_ref,
                wg1_ref, p_ref, hnei0_scr):
    p = pl.program_id(0)
    i = pl.program_id(1)

    @pl.when(p == 0)
    def _phase0():
        hnei0_scr[pl.ds(i * (NA // PG), NA // PG), :] = _elu(
            _dot(feat1_ref[...], w1_ref[...]) + b1_ref[...])

    @pl.when(p == 1)
    def _phase1():
        nei0 = nei0_ref[...]
        cnt0 = jnp.sum(nei0, axis=1, keepdims=True)
        cnt0 = jnp.where(cnt0 > 0, cnt0, 1.0)
        agg0 = _dot(nei0, hnei0_scr[...]) / cnt0
        hnei1 = _elu(_dot(feat2_ref[...], w2_ref[...]) + b2_ref[...])
        nei1 = nei1_ref[...]
        cnt1 = jnp.sum(nei1, axis=1, keepdims=True)
        cnt1 = jnp.where(cnt1 > 0, cnt1, 1.0)
        agg1 = _dot(nei1, hnei1) / cnt1
        h_tar = htar_ref[...]
        h_mask = hmask_ref[...]
        a0w = _dot(agg0, wagg0_ref[...])
        a1w = _dot(agg1, wagg1_ref[...])
        wg1 = wg1_ref[...]
        p_ref[:, 0 * D:1 * D] = _dot(h_tar, wg1)
        p_ref[:, 1 * D:2 * D] = _dot(_elu(h_tar + a0w), wg1)
        p_ref[:, 2 * D:3 * D] = _dot(_elu(h_mask + a0w), wg1)
        p_ref[:, 3 * D:4 * D] = _dot(_elu(h_tar + a1w), wg1)
        p_ref[:, 4 * D:5 * D] = _dot(_elu(h_mask + a1w), wg1)


def _gcn_kernel(adj0_ref, adj1_ref, madj0_ref, madj1_ref, p_ref, bg1_ref,
                wg2_ref, bg2_ref, z_ref, q_scr):
    ph = pl.program_id(0)
    i = pl.program_id(1)
    adj0 = adj0_ref[...]
    adj1 = adj1_ref[...]
    adjm = 0.5 * (adj0 + adj1)
    madj0 = madj0_ref[...]
    madj1 = madj1_ref[...]

    @pl.when(ph == 0)
    def _phase0():
        pmat = p_ref[...]
        bg1 = bg1_ref[...]
        wg2 = wg2_ref[...]
        r = pl.ds(i * (N // PG), N // PG)
        for v, adj in enumerate((adjm, adj0, madj0, adj1, madj1)):
            y = jax.nn.relu(_dot(adj, pmat[:, v * D:(v + 1) * D]) + bg1)
            q_scr[r, v * D:(v + 1) * D] = _dot(y, wg2)

    @pl.when(ph == 1)
    def _phase1():
        q = q_scr[...]
        bg2 = bg2_ref[...]
        for v, adj in enumerate((adjm, adj0, madj0, adj1, madj1)):
            z = _dot(adj, q[:, v * D:(v + 1) * D]) + bg2
            if v > 0:
                z = _normalize(z)
            z_ref[:, v * D:(v + 1) * D] = z


def _final_kernel(z_ref, watt_ref, batt_ref, aatt_ref, wproj_ref, bproj_ref,
                  wm1_ref, bm1_ref, wm2_ref, bm2_ref, out_ref):
    zmat = z_ref[...]
    z_coarse = zmat[:, 0 * D:1 * D]
    hf0 = zmat[:, 1 * D:2 * D]
    hf1 = zmat[:, 2 * D:3 * D]
    hf2 = zmat[:, 3 * D:4 * D]
    hf3 = zmat[:, 4 * D:5 * D]

    watt = watt_ref[...]
    batt = batt_ref[...]
    aatt = aatt_ref[...]

    def score(h):
        t = jnp.tanh(_dot(h, watt) + batt)
        return jnp.sum(_dot(t, aatt)) / N

    s0, s1, s2, s3 = score(hf0), score(hf1), score(hf2), score(hf3)
    m = jnp.maximum(jnp.maximum(s0, s1), jnp.maximum(s2, s3))
    e0, e1 = jnp.exp(s0 - m), jnp.exp(s1 - m)
    e2, e3 = jnp.exp(s2 - m), jnp.exp(s3 - m)
    tot = e0 + e1 + e2 + e3
    z_fine = (e0 * hf0 + e1 * hf1 + e2 * hf2 + e3 * hf3) / tot

    wproj = wproj_ref[...]
    bproj = bproj_ref[...]
    zc = _normalize(jnp.tanh(_dot(z_coarse, wproj) + bproj))
    zf = _normalize(jnp.tanh(_dot(z_fine, wproj) + bproj))

    s = _dot(zf, zc.T) * (1.0 / TAU)
    e = jnp.exp(s)
    a = _dot(zf, wm1_ref[...]) + bm1_ref[...]
    b = _dot(zc, wm1_ref[...])
    bt = b.T
    wm2 = wm2_ref[...]
    acc = jnp.full((N, N), bm2_ref[0, 0], dtype=jnp.float32)
    for k in range(16):
        acc = acc + jnp.tanh(a[:, k:k + 1] + bt[k:k + 1, :]) * wm2[0, k]
    weight = jax.nn.sigmoid(acc)
    den = jnp.sum(e * weight, axis=1)
    diag = jnp.sum(zf * zc, axis=1) * (1.0 / TAU)
    out_ref[...] = jnp.reshape(jnp.sum(jnp.log(den) - diag) / N, (1, 1))


def kernel(feat0, feat1, feat2, mask_feat, adj0, adj1, mask_adj0, mask_adj1,
           nei0, nei1, W_fc0, b_fc0, W_fc1, b_fc1, W_fc2, b_fc2, W_agg0,
           W_agg1, W_g1, b_g1, W_g2, b_g2, W_att, b_att, a_att, W_proj,
           b_proj, W_m1, b_m1, W_m2, b_m2):
    f32 = jnp.float32
    sds = jax.ShapeDtypeStruct

    h_tar, h_mask = pl.pallas_call(
        _encode_kernel,
        grid=(EG,),
        in_specs=[
            pl.BlockSpec((N // EG, F0), lambda i: (i, 0)),
            pl.BlockSpec((N // EG, F0), lambda i: (i, 0)),
            pl.BlockSpec((F0, H), lambda i: (0, 0)),
            pl.BlockSpec((1, H), lambda i: (0, 0)),
        ],
        out_specs=(pl.BlockSpec((N // EG, H), lambda i: (i, 0)),
                   pl.BlockSpec((N // EG, H), lambda i: (i, 0))),
        out_shape=(sds((N, H), f32), sds((N, H), f32)),
    )(feat0, mask_feat, W_fc0, b_fc0.reshape(1, H))

    tile = lambda r, c: pl.BlockSpec((r, c), lambda p, i: (i, 0))
    tile1 = lambda r, c: pl.BlockSpec((r, c), lambda p, i: (i * p, 0))
    const = lambda r, c: pl.BlockSpec((r, c), lambda p, i: (0, 0))

    p_mat = pl.pallas_call(
        _agg_kernel,
        grid=(2, PG),
        in_specs=[
            pl.BlockSpec((NA // PG, F1),
                         lambda p, i: (jnp.where(p == 0, i, PG - 1), 0)),
            const(F1, H),
            const(1, H),
            tile1(N // PG, NA),
            tile1(N // PG, NS),
            const(NS, F2),
            const(F2, H),
            const(1, H),
            tile1(N // PG, H),
            tile1(N // PG, H),
            const(H, H),
            const(H, H),
            const(H, D),
        ],
        out_specs=pl.BlockSpec((N // PG, 5 * D), lambda p, i: (i, 0)),
        out_shape=sds((N, 5 * D), f32),
        scratch_shapes=[pltpu.VMEM((NA, H), f32)],
    )(feat1, W_fc1, b_fc1.reshape(1, H), nei0, nei1, feat2, W_fc2,
      b_fc2.reshape(1, H), h_tar, h_mask, W_agg0, W_agg1, W_g1)

    z_mat = pl.pallas_call(
        _gcn_kernel,
        grid=(2, PG),
        in_specs=[
            tile(N // PG, N),
            tile(N // PG, N),
            tile(N // PG, N),
            tile(N // PG, N),
            const(N, 5 * D),
            const(1, D),
            const(D, D),
            const(1, D),
        ],
        out_specs=pl.BlockSpec((N // PG, 5 * D), lambda p, i: (i, 0)),
        out_shape=sds((N, 5 * D), f32),
        scratch_shapes=[pltpu.VMEM((N, 5 * D), f32)],
    )(adj0, adj1, mask_adj0, mask_adj1, p_mat, b_g1.reshape(1, D), W_g2,
      b_g2.reshape(1, D))

    loss = pl.pallas_call(
        _final_kernel,
        out_shape=sds((1, 1), f32),
    )(z_mat, W_att, b_att.reshape(1, D), a_att.reshape(D, 1), W_proj,
      b_proj.reshape(1, D), W_m1, b_m1.reshape(1, 16), W_m2.reshape(1, 16),
      b_m2.reshape(1, 1))
    return loss[0, 0]


# speedup vs baseline: 1.0686x; 1.0686x over previous
import jax
import jax.numpy as jnp
from jax.experimental import pallas as pl
from jax.experimental.pallas import tpu as pltpu

N, NA, NS = 1024, 4096, 60
F0, F1, F2 = 1902, 334, 64
H, D = 256, 64
TAU = 0.5

EG = 4
PG = 4


def _elu(x):
    return jnp.where(x > 0, x, jnp.exp(x) - 1.0)


def _normalize(x):
    nrm = jnp.sqrt(jnp.sum(x * x, axis=1, keepdims=True))
    return x / jnp.clip(nrm, 1e-12)


def _dot(a, b):
    return jnp.dot(a, b, preferred_element_type=jnp.float32)


def _encode_kernel(feat0_ref, mask_ref, w0_ref, b0_ref, htar_ref, hmask_ref):
    w0 = w0_ref[...]
    b0 = b0_ref[...]
    htar_ref[...] = _elu(_dot(feat0_ref[...], w0) + b0)
    hmask_ref[...] = _elu(_dot(mask_ref[...], w0) + b0)


def _agg_kernel(feat1_ref, w1_ref, b1_ref, nei0_ref, nei1_ref, feat2_ref,
                w2_ref, b2_ref, htar_ref, hmask_ref, wagg0_ref, wagg1_ref,
                wg1_ref, p_ref, hnei0_scr):
    p = pl.program_id(0)
    i = pl.program_id(1)

    @pl.when(p == 0)
    def _phase0():
        hnei0_scr[pl.ds(i * (NA // PG), NA // PG), :] = _elu(
            _dot(feat1_ref[...], w1_ref[...]) + b1_ref[...])

    @pl.when(p == 1)
    def _phase1():
        nei0 = nei0_ref[...]
        cnt0 = jnp.sum(nei0, axis=1, keepdims=True)
        cnt0 = jnp.where(cnt0 > 0, cnt0, 1.0)
        agg0 = _dot(nei0, hnei0_scr[...]) / cnt0
        hnei1 = _elu(_dot(feat2_ref[...], w2_ref[...]) + b2_ref[...])
        nei1 = nei1_ref[...]
        cnt1 = jnp.sum(nei1, axis=1, keepdims=True)
        cnt1 = jnp.where(cnt1 > 0, cnt1, 1.0)
        agg1 = _dot(nei1, hnei1) / cnt1
        h_tar = htar_ref[...]
        h_mask = hmask_ref[...]
        a0w = _dot(agg0, wagg0_ref[...])
        a1w = _dot(agg1, wagg1_ref[...])
        wg1 = wg1_ref[...]
        p_ref[:, 0 * D:1 * D] = _dot(h_tar, wg1)
        p_ref[:, 1 * D:2 * D] = _dot(_elu(h_tar + a0w), wg1)
        p_ref[:, 2 * D:3 * D] = _dot(_elu(h_mask + a0w), wg1)
        p_ref[:, 3 * D:4 * D] = _dot(_elu(h_tar + a1w), wg1)
        p_ref[:, 4 * D:5 * D] = _dot(_elu(h_mask + a1w), wg1)


def _gcn_kernel(adj0_ref, adj1_ref, madj0_ref, madj1_ref, p_ref, bg1_ref,
                wg2_ref, bg2_ref, z_ref, adj_scr, q_scr):
    i = pl.program_id(0)

    @pl.when(i < PG)
    def _stream():
        adj0 = adj0_ref[...]
        adj1 = adj1_ref[...]
        madj0 = madj0_ref[...]
        madj1 = madj1_ref[...]
        r = pl.ds(i * (N // PG), N // PG)
        adj_scr[0, r, :] = adj0
        adj_scr[1, r, :] = adj1
        adj_scr[2, r, :] = madj0
        adj_scr[3, r, :] = madj1
        pmat = p_ref[...]
        bg1 = bg1_ref[...]
        wg2 = wg2_ref[...]
        p0 = pmat[:, 0:D]
        ym = jax.nn.relu(0.5 * (_dot(adj0, p0) + _dot(adj1, p0)) + bg1)
        q_scr[r, 0:D] = _dot(ym, wg2)
        for v, adj in ((1, adj0), (2, madj0), (3, adj1), (4, madj1)):
            y = jax.nn.relu(_dot(adj, pmat[:, v * D:(v + 1) * D]) + bg1)
            q_scr[r, v * D:(v + 1) * D] = _dot(y, wg2)

    @pl.when(i == PG)
    def _finish():
        q = q_scr[...]
        bg2 = bg2_ref[...]
        q0 = q[:, 0:D]
        for t in range(PG):
            r = pl.ds(t * (N // PG), N // PG)
            a0 = adj_scr[0, r, :]
            a1 = adj_scr[1, r, :]
            z_ref[r, 0:D] = 0.5 * (_dot(a0, q0) + _dot(a1, q0)) + bg2
            z_ref[r, D:2 * D] = _normalize(_dot(a0, q[:, D:2 * D]) + bg2)
            z_ref[r, 2 * D:3 * D] = _normalize(
                _dot(adj_scr[2, r, :], q[:, 2 * D:3 * D]) + bg2)
            z_ref[r, 3 * D:4 * D] = _normalize(_dot(a1, q[:, 3 * D:4 * D]) + bg2)
            z_ref[r, 4 * D:5 * D] = _normalize(
                _dot(adj_scr[3, r, :], q[:, 4 * D:5 * D]) + bg2)


def _final_kernel(z_ref, watt_ref, batt_ref, aatt_ref, wproj_ref, bproj_ref,
                  wm1_ref, bm1_ref, wm2_ref, bm2_ref, out_ref):
    zmat = z_ref[...]
    z_coarse = zmat[:, 0:D]
    hf0 = zmat[:, D:2 * D]
    hf1 = zmat[:, 2 * D:3 * D]
    hf2 = zmat[:, 3 * D:4 * D]
    hf3 = zmat[:, 4 * D:5 * D]

    watt = watt_ref[...]
    batt = batt_ref[...]
    aatt = aatt_ref[...]

    def score(h):
        t = jnp.tanh(_dot(h, watt) + batt)
        return jnp.sum(_dot(t, aatt)) / N

    s0, s1, s2, s3 = score(hf0), score(hf1), score(hf2), score(hf3)
    m = jnp.maximum(jnp.maximum(s0, s1), jnp.maximum(s2, s3))
    e0, e1 = jnp.exp(s0 - m), jnp.exp(s1 - m)
    e2, e3 = jnp.exp(s2 - m), jnp.exp(s3 - m)
    tot = e0 + e1 + e2 + e3
    z_fine = (e0 * hf0 + e1 * hf1 + e2 * hf2 + e3 * hf3) / tot

    wproj = wproj_ref[...]
    bproj = bproj_ref[...]
    zc = _normalize(jnp.tanh(_dot(z_coarse, wproj) + bproj))
    zf = _normalize(jnp.tanh(_dot(z_fine, wproj) + bproj))

    zct = zc.T
    a = _dot(zf, wm1_ref[...]) + bm1_ref[...]
    bt = _dot(zc, wm1_ref[...]).T
    wm2 = wm2_ref[...]
    bm2 = bm2_ref[0, 0]
    T = 128
    total = jnp.float32(0.0)
    for t in range(N // T):
        r = slice(t * T, (t + 1) * T)
        zf_t = zf[r]
        e_t = jnp.exp(_dot(zf_t, zct) * (1.0 / TAU))
        a_t = a[r]
        acc = jnp.full((T, N), bm2, dtype=jnp.float32)
        for k in range(16):
            acc = acc + jnp.tanh(a_t[:, k:k + 1] + bt[k:k + 1, :]) * wm2[0, k]
        den_t = jnp.sum(e_t * jax.nn.sigmoid(acc), axis=1)
        diag_t = jnp.sum(zf_t * zc[r], axis=1) * (1.0 / TAU)
        total = total + jnp.sum(jnp.log(den_t) - diag_t)
    out_ref[...] = jnp.reshape(total / N, (1, 1))


def kernel(feat0, feat1, feat2, mask_feat, adj0, adj1, mask_adj0, mask_adj1,
           nei0, nei1, W_fc0, b_fc0, W_fc1, b_fc1, W_fc2, b_fc2, W_agg0,
           W_agg1, W_g1, b_g1, W_g2, b_g2, W_att, b_att, a_att, W_proj,
           b_proj, W_m1, b_m1, W_m2, b_m2):
    f32 = jnp.float32
    sds = jax.ShapeDtypeStruct

    h_tar, h_mask = pl.pallas_call(
        _encode_kernel,
        grid=(EG,),
        in_specs=[
            pl.BlockSpec((N // EG, F0), lambda i: (i, 0)),
            pl.BlockSpec((N // EG, F0), lambda i: (i, 0)),
            pl.BlockSpec((F0, H), lambda i: (0, 0)),
            pl.BlockSpec((1, H), lambda i: (0, 0)),
        ],
        out_specs=(pl.BlockSpec((N // EG, H), lambda i: (i, 0)),
                   pl.BlockSpec((N // EG, H), lambda i: (i, 0))),
        out_shape=(sds((N, H), f32), sds((N, H), f32)),
    )(feat0, mask_feat, W_fc0, b_fc0.reshape(1, H))

    tile = lambda r, c: pl.BlockSpec((r, c), lambda p, i: (i, 0))
    tile1 = lambda r, c: pl.BlockSpec((r, c), lambda p, i: (i * p, 0))
    const = lambda r, c: pl.BlockSpec((r, c), lambda p, i: (0, 0))

    p_mat = pl.pallas_call(
        _agg_kernel,
        grid=(2, PG),
        in_specs=[
            pl.BlockSpec((NA // PG, F1),
                         lambda p, i: (jnp.where(p == 0, i, PG - 1), 0)),
            const(F1, H),
            const(1, H),
            tile1(N // PG, NA),
            tile1(N // PG, NS),
            const(NS, F2),
            const(F2, H),
            const(1, H),
            tile1(N // PG, H),
            tile1(N // PG, H),
            const(H, H),
            const(H, H),
            const(H, D),
        ],
        out_specs=pl.BlockSpec((N // PG, 5 * D), lambda p, i: (i, 0)),
        out_shape=sds((N, 5 * D), f32),
        scratch_shapes=[pltpu.VMEM((NA, H), f32)],
    )(feat1, W_fc1, b_fc1.reshape(1, H), nei0, nei1, feat2, W_fc2,
      b_fc2.reshape(1, H), h_tar, h_mask, W_agg0, W_agg1, W_g1)

    stile = lambda: pl.BlockSpec((N // PG, N),
                                 lambda i: (jnp.minimum(i, PG - 1), 0))
    cst = lambda r, c: pl.BlockSpec((r, c), lambda i: (0, 0))

    z_mat = pl.pallas_call(
        _gcn_kernel,
        grid=(PG + 1,),
        in_specs=[
            stile(),
            stile(),
            stile(),
            stile(),
            cst(N, 5 * D),
            cst(1, D),
            cst(D, D),
            cst(1, D),
        ],
        out_specs=pl.BlockSpec((N, 5 * D), lambda i: (0, 0)),
        out_shape=sds((N, 5 * D), f32),
        scratch_shapes=[pltpu.VMEM((4, N, N), f32),
                        pltpu.VMEM((N, 5 * D), f32)],
    )(adj0, adj1, mask_adj0, mask_adj1, p_mat, b_g1.reshape(1, D), W_g2,
      b_g2.reshape(1, D))

    loss = pl.pallas_call(
        _final_kernel,
        out_shape=sds((1, 1), f32),
    )(z_mat, W_att, b_att.reshape(1, D), a_att.reshape(D, 1), W_proj,
      b_proj.reshape(1, D), W_m1, b_m1.reshape(1, 16), W_m2.reshape(1, 16),
      b_m2.reshape(1, 1))
    return loss[0, 0]


# speedup vs baseline: 1.0828x; 1.0132x over previous
import jax
import jax.numpy as jnp
from jax.experimental import pallas as pl
from jax.experimental.pallas import tpu as pltpu

N, NA, NS = 1024, 4096, 60
F0, F1, F2 = 1902, 334, 64
H, D = 256, 64
TAU = 0.5

EG = 4
PG = 4


def _elu(x):
    return jnp.where(x > 0, x, jnp.exp(x) - 1.0)


def _normalize(x):
    nrm = jnp.sqrt(jnp.sum(x * x, axis=1, keepdims=True))
    return x / jnp.clip(nrm, 1e-12)


def _dot(a, b):
    return jnp.dot(a, b, preferred_element_type=jnp.float32)


def _front_kernel(feat0_ref, mask_ref, w0_ref, b0_ref, feat1_ref, w1_ref,
                  b1_ref, nei0_ref, nei1_ref, feat2_ref, w2_ref, b2_ref,
                  wagg0_ref, wagg1_ref, wg1_ref, p_ref, htar_scr, hmask_scr,
                  hnei0_scr):
    p = pl.program_id(0)
    i = pl.program_id(1)

    @pl.when(p == 0)
    def _phasee():
        w0 = w0_ref[...]
        b0 = b0_ref[...]
        r = pl.ds(i * (N // PG), N // PG)
        htar_scr[r, :] = _elu(_dot(feat0_ref[...], w0) + b0)
        hmask_scr[r, :] = _elu(_dot(mask_ref[...], w0) + b0)

    @pl.when(p == 1)
    def _phase0():
        hnei0_scr[pl.ds(i * (NA // PG), NA // PG), :] = _elu(
            _dot(feat1_ref[...], w1_ref[...]) + b1_ref[...])

    @pl.when(p == 2)
    def _phase1():
        nei0 = nei0_ref[...]
        cnt0 = jnp.sum(nei0, axis=1, keepdims=True)
        cnt0 = jnp.where(cnt0 > 0, cnt0, 1.0)
        agg0 = _dot(nei0, hnei0_scr[...]) / cnt0
        hnei1 = _elu(_dot(feat2_ref[...], w2_ref[...]) + b2_ref[...])
        nei1 = nei1_ref[...]
        cnt1 = jnp.sum(nei1, axis=1, keepdims=True)
        cnt1 = jnp.where(cnt1 > 0, cnt1, 1.0)
        agg1 = _dot(nei1, hnei1) / cnt1
        r = pl.ds(i * (N // PG), N // PG)
        h_tar = htar_scr[r, :]
        h_mask = hmask_scr[r, :]
        a0w = _dot(agg0, wagg0_ref[...])
        a1w = _dot(agg1, wagg1_ref[...])
        wg1 = wg1_ref[...]
        p_ref[:, 0 * D:1 * D] = _dot(h_tar, wg1)
        p_ref[:, 1 * D:2 * D] = _dot(_elu(h_tar + a0w), wg1)
        p_ref[:, 2 * D:3 * D] = _dot(_elu(h_mask + a0w), wg1)
        p_ref[:, 3 * D:4 * D] = _dot(_elu(h_tar + a1w), wg1)
        p_ref[:, 4 * D:5 * D] = _dot(_elu(h_mask + a1w), wg1)


def _gcn_kernel(adj0_ref, adj1_ref, madj0_ref, madj1_ref, p_ref, bg1_ref,
                wg2_ref, bg2_ref, z_ref, adj_scr, q_scr):
    i = pl.program_id(0)

    @pl.when(i < PG)
    def _stream():
        adj0 = adj0_ref[...]
        adj1 = adj1_ref[...]
        madj0 = madj0_ref[...]
        madj1 = madj1_ref[...]
        r = pl.ds(i * (N // PG), N // PG)
        adj_scr[0, r, :] = adj0
        adj_scr[1, r, :] = adj1
        adj_scr[2, r, :] = madj0
        adj_scr[3, r, :] = madj1
        pmat = p_ref[...]
        bg1 = bg1_ref[...]
        wg2 = wg2_ref[...]
        p0 = pmat[:, 0:D]
        ym = jax.nn.relu(0.5 * (_dot(adj0, p0) + _dot(adj1, p0)) + bg1)
        q_scr[r, 0:D] = _dot(ym, wg2)
        for v, adj in ((1, adj0), (2, madj0), (3, adj1), (4, madj1)):
            y = jax.nn.relu(_dot(adj, pmat[:, v * D:(v + 1) * D]) + bg1)
            q_scr[r, v * D:(v + 1) * D] = _dot(y, wg2)

    @pl.when(i == PG)
    def _finish():
        q = q_scr[...]
        bg2 = bg2_ref[...]
        q0 = q[:, 0:D]
        for t in range(PG):
            r = pl.ds(t * (N // PG), N // PG)
            a0 = adj_scr[0, r, :]
            a1 = adj_scr[1, r, :]
            z_ref[r, 0:D] = 0.5 * (_dot(a0, q0) + _dot(a1, q0)) + bg2
            z_ref[r, D:2 * D] = _normalize(_dot(a0, q[:, D:2 * D]) + bg2)
            z_ref[r, 2 * D:3 * D] = _normalize(
                _dot(adj_scr[2, r, :], q[:, 2 * D:3 * D]) + bg2)
            z_ref[r, 3 * D:4 * D] = _normalize(_dot(a1, q[:, 3 * D:4 * D]) + bg2)
            z_ref[r, 4 * D:5 * D] = _normalize(
                _dot(adj_scr[3, r, :], q[:, 4 * D:5 * D]) + bg2)


def _final_kernel(z_ref, watt_ref, batt_ref, aatt_ref, wproj_ref, bproj_ref,
                  wm1_ref, bm1_ref, wm2_ref, bm2_ref, out_ref):
    zmat = z_ref[...]
    z_coarse = zmat[:, 0:D]
    hf0 = zmat[:, D:2 * D]
    hf1 = zmat[:, 2 * D:3 * D]
    hf2 = zmat[:, 3 * D:4 * D]
    hf3 = zmat[:, 4 * D:5 * D]

    watt = watt_ref[...]
    batt = batt_ref[...]
    aatt = aatt_ref[...]

    def score(h):
        t = jnp.tanh(_dot(h, watt) + batt)
        return jnp.sum(_dot(t, aatt)) / N

    s0, s1, s2, s3 = score(hf0), score(hf1), score(hf2), score(hf3)
    m = jnp.maximum(jnp.maximum(s0, s1), jnp.maximum(s2, s3))
    e0, e1 = jnp.exp(s0 - m), jnp.exp(s1 - m)
    e2, e3 = jnp.exp(s2 - m), jnp.exp(s3 - m)
    tot = e0 + e1 + e2 + e3
    z_fine = (e0 * hf0 + e1 * hf1 + e2 * hf2 + e3 * hf3) / tot

    wproj = wproj_ref[...]
    bproj = bproj_ref[...]
    zc = _normalize(jnp.tanh(_dot(z_coarse, wproj) + bproj))
    zf = _normalize(jnp.tanh(_dot(z_fine, wproj) + bproj))

    zct = zc.T
    a = _dot(zf, wm1_ref[...]) + bm1_ref[...]
    bt = _dot(zc, wm1_ref[...]).T
    wm2 = wm2_ref[...]
    bm2 = bm2_ref[0, 0]
    T = 128
    total = jnp.float32(0.0)
    for t in range(N // T):
        r = slice(t * T, (t + 1) * T)
        zf_t = zf[r]
        e_t = jnp.exp(_dot(zf_t, zct) * (1.0 / TAU))
        a_t = a[r]
        acc = jnp.full((T, N), bm2, dtype=jnp.float32)
        for k in range(16):
            acc = acc + jnp.tanh(a_t[:, k:k + 1] + bt[k:k + 1, :]) * wm2[0, k]
        den_t = jnp.sum(e_t * jax.nn.sigmoid(acc), axis=1)
        diag_t = jnp.sum(zf_t * zc[r], axis=1) * (1.0 / TAU)
        total = total + jnp.sum(jnp.log(den_t) - diag_t)
    out_ref[...] = jnp.reshape(total / N, (1, 1))


def kernel(feat0, feat1, feat2, mask_feat, adj0, adj1, mask_adj0, mask_adj1,
           nei0, nei1, W_fc0, b_fc0, W_fc1, b_fc1, W_fc2, b_fc2, W_agg0,
           W_agg1, W_g1, b_g1, W_g2, b_g2, W_att, b_att, a_att, W_proj,
           b_proj, W_m1, b_m1, W_m2, b_m2):
    f32 = jnp.float32
    sds = jax.ShapeDtypeStruct

    const = lambda r, c: pl.BlockSpec((r, c), lambda p, i: (0, 0))
    ph0 = lambda r, c: pl.BlockSpec(
        (r, c), lambda p, i: (jnp.where(p == 0, i, PG - 1), 0))
    ph1 = lambda r, c: pl.BlockSpec(
        (r, c), lambda p, i: (jnp.where(p == 1, i, jnp.where(p == 0, 0, PG - 1)), 0))
    ph2 = lambda r, c: pl.BlockSpec(
        (r, c), lambda p, i: (jnp.where(p == 2, i, 0), 0))

    p_mat = pl.pallas_call(
        _front_kernel,
        grid=(3, PG),
        in_specs=[
            ph0(N // PG, F0),
            ph0(N // PG, F0),
            const(F0, H),
            const(1, H),
            ph1(NA // PG, F1),
            const(F1, H),
            const(1, H),
            ph2(N // PG, NA),
            ph2(N // PG, NS),
            const(NS, F2),
            const(F2, H),
            const(1, H),
            const(H, H),
            const(H, H),
            const(H, D),
        ],
        out_specs=pl.BlockSpec((N // PG, 5 * D), lambda p, i: (i, 0)),
        out_shape=sds((N, 5 * D), f32),
        scratch_shapes=[pltpu.VMEM((N, H), f32),
                        pltpu.VMEM((N, H), f32),
                        pltpu.VMEM((NA, H), f32)],
    )(feat0, mask_feat, W_fc0, b_fc0.reshape(1, H), feat1, W_fc1,
      b_fc1.reshape(1, H), nei0, nei1, feat2, W_fc2, b_fc2.reshape(1, H),
      W_agg0, W_agg1, W_g1)

    stile = lambda: pl.BlockSpec((N // PG, N),
                                 lambda i: (jnp.minimum(i, PG - 1), 0))
    cst = lambda r, c: pl.BlockSpec((r, c), lambda i: (0, 0))

    z_mat = pl.pallas_call(
        _gcn_kernel,
        grid=(PG + 1,),
        in_specs=[
            stile(),
            stile(),
            stile(),
            stile(),
            cst(N, 5 * D),
            cst(1, D),
            cst(D, D),
            cst(1, D),
        ],
        out_specs=pl.BlockSpec((N, 5 * D), lambda i: (0, 0)),
        out_shape=sds((N, 5 * D), f32),
        scratch_shapes=[pltpu.VMEM((4, N, N), f32),
                        pltpu.VMEM((N, 5 * D), f32)],
    )(adj0, adj1, mask_adj0, mask_adj1, p_mat, b_g1.reshape(1, D), W_g2,
      b_g2.reshape(1, D))

    loss = pl.pallas_call(
        _final_kernel,
        out_shape=sds((1, 1), f32),
    )(z_mat, W_att, b_att.reshape(1, D), a_att.reshape(D, 1), W_proj,
      b_proj.reshape(1, D), W_m1, b_m1.reshape(1, 16), W_m2.reshape(1, 16),
      b_m2.reshape(1, 1))
    return loss[0, 0]


# speedup vs baseline: 1.1097x; 1.0249x over previous
import jax
import jax.numpy as jnp
from jax.experimental import pallas as pl
from jax.experimental.pallas import tpu as pltpu

N, NA, NS = 1024, 4096, 60
F0, F1, F2 = 1902, 334, 64
H, D = 256, 64
TAU = 0.5

EG = 4
PG = 4


def _elu(x):
    return jnp.where(x > 0, x, jnp.exp(x) - 1.0)


def _normalize(x):
    nrm = jnp.sqrt(jnp.sum(x * x, axis=1, keepdims=True))
    return x / jnp.clip(nrm, 1e-12)


def _dot(a, b):
    return jnp.dot(a, b, preferred_element_type=jnp.float32)


def _front_kernel(feat0_ref, mask_ref, w0_ref, b0_ref, feat1_ref, w1_ref,
                  b1_ref, nei0_ref, nei1_ref, feat2_ref, w2_ref, b2_ref,
                  wagg0_ref, wagg1_ref, wg1_ref, p_ref, htar_scr, hmask_scr,
                  hnei0_scr):
    p = pl.program_id(0)
    i = pl.program_id(1)

    @pl.when(p == 0)
    def _phasee():
        w0 = w0_ref[...]
        b0 = b0_ref[...]
        r = pl.ds(i * (N // PG), N // PG)
        htar_scr[r, :] = _elu(_dot(feat0_ref[...], w0) + b0)
        hmask_scr[r, :] = _elu(_dot(mask_ref[...], w0) + b0)

    @pl.when(p == 1)
    def _phase0():
        hnei0_scr[pl.ds(i * (NA // PG), NA // PG), :] = _elu(
            _dot(feat1_ref[...], w1_ref[...]) + b1_ref[...])

    @pl.when(p == 2)
    def _phase1():
        nei0 = nei0_ref[...]
        cnt0 = jnp.sum(nei0, axis=1, keepdims=True)
        cnt0 = jnp.where(cnt0 > 0, cnt0, 1.0)
        agg0 = _dot(nei0, hnei0_scr[...]) / cnt0
        hnei1 = _elu(_dot(feat2_ref[...], w2_ref[...]) + b2_ref[...])
        nei1 = nei1_ref[...]
        cnt1 = jnp.sum(nei1, axis=1, keepdims=True)
        cnt1 = jnp.where(cnt1 > 0, cnt1, 1.0)
        agg1 = _dot(nei1, hnei1) / cnt1
        r = pl.ds(i * (N // PG), N // PG)
        h_tar = htar_scr[r, :]
        h_mask = hmask_scr[r, :]
        a0w = _dot(agg0, wagg0_ref[...])
        a1w = _dot(agg1, wagg1_ref[...])
        wg1 = wg1_ref[...]
        p_ref[:, 0 * D:1 * D] = _dot(h_tar, wg1)
        p_ref[:, 1 * D:2 * D] = _dot(_elu(h_tar + a0w), wg1)
        p_ref[:, 2 * D:3 * D] = _dot(_elu(h_mask + a0w), wg1)
        p_ref[:, 3 * D:4 * D] = _dot(_elu(h_tar + a1w), wg1)
        p_ref[:, 4 * D:5 * D] = _dot(_elu(h_mask + a1w), wg1)


def _gcn_kernel(adj0_ref, adj1_ref, madj0_ref, madj1_ref, p_ref, bg1_ref,
                wg2_ref, bg2_ref, z_ref, adj_scr, q_scr):
    i = pl.program_id(0)

    @pl.when(i < PG)
    def _stream():
        adj0 = adj0_ref[...]
        adj1 = adj1_ref[...]
        madj0 = madj0_ref[...]
        madj1 = madj1_ref[...]
        r = pl.ds(i * (N // PG), N // PG)
        adj_scr[0, r, :] = adj0
        adj_scr[1, r, :] = adj1
        adj_scr[2, r, :] = madj0
        adj_scr[3, r, :] = madj1
        pmat = p_ref[...]
        bg1 = bg1_ref[...]
        wg2 = wg2_ref[...]
        p0 = pmat[:, 0:D]
        ym = jax.nn.relu(0.5 * (_dot(adj0, p0) + _dot(adj1, p0)) + bg1)
        q_scr[r, 0:D] = _dot(ym, wg2)
        for v, adj in ((1, adj0), (2, madj0), (3, adj1), (4, madj1)):
            y = jax.nn.relu(_dot(adj, pmat[:, v * D:(v + 1) * D]) + bg1)
            q_scr[r, v * D:(v + 1) * D] = _dot(y, wg2)

    @pl.when(i == PG)
    def _finish():
        q = q_scr[...]
        bg2 = bg2_ref[...]
        q0 = q[:, 0:D]
        for t in range(PG):
            r = pl.ds(t * (N // PG), N // PG)
            a0 = adj_scr[0, r, :]
            a1 = adj_scr[1, r, :]
            z_ref[r, 0:D] = 0.5 * (_dot(a0, q0) + _dot(a1, q0)) + bg2
            z_ref[r, D:2 * D] = _normalize(_dot(a0, q[:, D:2 * D]) + bg2)
            z_ref[r, 2 * D:3 * D] = _normalize(
                _dot(adj_scr[2, r, :], q[:, 2 * D:3 * D]) + bg2)
            z_ref[r, 3 * D:4 * D] = _normalize(_dot(a1, q[:, 3 * D:4 * D]) + bg2)
            z_ref[r, 4 * D:5 * D] = _normalize(
                _dot(adj_scr[3, r, :], q[:, 4 * D:5 * D]) + bg2)


def _final_kernel(z_ref, watt_ref, batt_ref, aatt_ref, wproj_ref, bproj_ref,
                  wm1_ref, bm1_ref, wm2_ref, bm2_ref, out_ref):
    zmat = z_ref[...]
    z_coarse = zmat[:, 0:D]
    hf0 = zmat[:, D:2 * D]
    hf1 = zmat[:, 2 * D:3 * D]
    hf2 = zmat[:, 3 * D:4 * D]
    hf3 = zmat[:, 4 * D:5 * D]

    watt = watt_ref[...]
    batt = batt_ref[...]
    aatt = aatt_ref[...]

    def score(h):
        t = jnp.tanh(_dot(h, watt) + batt)
        return jnp.sum(_dot(t, aatt)) / N

    s0, s1, s2, s3 = score(hf0), score(hf1), score(hf2), score(hf3)
    m = jnp.maximum(jnp.maximum(s0, s1), jnp.maximum(s2, s3))
    e0, e1 = jnp.exp(s0 - m), jnp.exp(s1 - m)
    e2, e3 = jnp.exp(s2 - m), jnp.exp(s3 - m)
    tot = e0 + e1 + e2 + e3
    z_fine = (e0 * hf0 + e1 * hf1 + e2 * hf2 + e3 * hf3) / tot

    wproj = wproj_ref[...]
    bproj = bproj_ref[...]
    zc = _normalize(jnp.tanh(_dot(z_coarse, wproj) + bproj))
    zf = _normalize(jnp.tanh(_dot(z_fine, wproj) + bproj))

    bf16 = jnp.bfloat16
    zct = zc.T
    a = (_dot(zf, wm1_ref[...]) + bm1_ref[...]).astype(bf16)
    bt = _dot(zc, wm1_ref[...]).T.astype(bf16)
    wm2 = wm2_ref[...].astype(bf16)
    bm2 = bm2_ref[0, 0]
    T = 128
    total = jnp.float32(0.0)
    for t in range(N // T):
        r = slice(t * T, (t + 1) * T)
        zf_t = zf[r]
        e_t = jnp.exp(_dot(zf_t, zct) * (1.0 / TAU))
        a_t = a[r]
        acc = jnp.full((T, N), bm2, dtype=bf16)
        for k in range(16):
            acc = acc + jnp.tanh(a_t[:, k:k + 1] + bt[k:k + 1, :]) * wm2[:, k:k + 1]
        den_t = jnp.sum(e_t * jax.nn.sigmoid(acc.astype(jnp.float32)), axis=1)
        diag_t = jnp.sum(zf_t * zc[r], axis=1) * (1.0 / TAU)
        total = total + jnp.sum(jnp.log(den_t) - diag_t)
    out_ref[...] = jnp.reshape(total / N, (1, 1))


def kernel(feat0, feat1, feat2, mask_feat, adj0, adj1, mask_adj0, mask_adj1,
           nei0, nei1, W_fc0, b_fc0, W_fc1, b_fc1, W_fc2, b_fc2, W_agg0,
           W_agg1, W_g1, b_g1, W_g2, b_g2, W_att, b_att, a_att, W_proj,
           b_proj, W_m1, b_m1, W_m2, b_m2):
    f32 = jnp.float32
    sds = jax.ShapeDtypeStruct

    const = lambda r, c: pl.BlockSpec((r, c), lambda p, i: (0, 0))
    ph0 = lambda r, c: pl.BlockSpec(
        (r, c), lambda p, i: (jnp.where(p == 0, i, PG - 1), 0))
    ph1 = lambda r, c: pl.BlockSpec(
        (r, c), lambda p, i: (jnp.where(p == 1, i, jnp.where(p == 0, 0, PG - 1)), 0))
    ph2 = lambda r, c: pl.BlockSpec(
        (r, c), lambda p, i: (jnp.where(p == 2, i, 0), 0))

    p_mat = pl.pallas_call(
        _front_kernel,
        grid=(3, PG),
        in_specs=[
            ph0(N // PG, F0),
            ph0(N // PG, F0),
            const(F0, H),
            const(1, H),
            ph1(NA // PG, F1),
            const(F1, H),
            const(1, H),
            ph2(N // PG, NA),
            ph2(N // PG, NS),
            const(NS, F2),
            const(F2, H),
            const(1, H),
            const(H, H),
            const(H, H),
            const(H, D),
        ],
        out_specs=pl.BlockSpec((N // PG, 5 * D), lambda p, i: (i, 0)),
        out_shape=sds((N, 5 * D), f32),
        scratch_shapes=[pltpu.VMEM((N, H), f32),
                        pltpu.VMEM((N, H), f32),
                        pltpu.VMEM((NA, H), f32)],
    )(feat0, mask_feat, W_fc0, b_fc0.reshape(1, H), feat1, W_fc1,
      b_fc1.reshape(1, H), nei0, nei1, feat2, W_fc2, b_fc2.reshape(1, H),
      W_agg0, W_agg1, W_g1)

    stile = lambda: pl.BlockSpec((N // PG, N),
                                 lambda i: (jnp.minimum(i, PG - 1), 0))
    cst = lambda r, c: pl.BlockSpec((r, c), lambda i: (0, 0))

    z_mat = pl.pallas_call(
        _gcn_kernel,
        grid=(PG + 1,),
        in_specs=[
            stile(),
            stile(),
            stile(),
            stile(),
            cst(N, 5 * D),
            cst(1, D),
            cst(D, D),
            cst(1, D),
        ],
        out_specs=pl.BlockSpec((N, 5 * D), lambda i: (0, 0)),
        out_shape=sds((N, 5 * D), f32),
        scratch_shapes=[pltpu.VMEM((4, N, N), f32),
                        pltpu.VMEM((N, 5 * D), f32)],
    )(adj0, adj1, mask_adj0, mask_adj1, p_mat, b_g1.reshape(1, D), W_g2,
      b_g2.reshape(1, D))

    loss = pl.pallas_call(
        _final_kernel,
        out_shape=sds((1, 1), f32),
    )(z_mat, W_att, b_att.reshape(1, D), a_att.reshape(D, 1), W_proj,
      b_proj.reshape(1, D), W_m1, b_m1.reshape(1, 16), W_m2.reshape(1, 16),
      b_m2.reshape(1, 1))
    return loss[0, 0]


# speedup vs baseline: 1.1340x; 1.0218x over previous
import jax
import jax.numpy as jnp
from jax.experimental import pallas as pl
from jax.experimental.pallas import tpu as pltpu

N, NA, NS = 1024, 4096, 60
F0, F1, F2 = 1902, 334, 64
H, D = 256, 64
TAU = 0.5

EG = 4
PG = 4


def _elu(x):
    return jnp.where(x > 0, x, jnp.exp(x) - 1.0)


def _normalize(x):
    nrm = jnp.sqrt(jnp.sum(x * x, axis=1, keepdims=True))
    return x / jnp.clip(nrm, 1e-12)


def _dot(a, b):
    return jnp.dot(a, b, preferred_element_type=jnp.float32)


def _bdot(a, b):
    return jnp.dot(a.astype(jnp.bfloat16), b.astype(jnp.bfloat16),
                   preferred_element_type=jnp.float32)


def _front_kernel(feat0_ref, mask_ref, w0_ref, b0_ref, feat1_ref, w1_ref,
                  b1_ref, nei0_ref, nei1_ref, feat2_ref, w2_ref, b2_ref,
                  wagg0_ref, wagg1_ref, wg1_ref, p_ref, htar_scr, hmask_scr,
                  hnei0_scr):
    p = pl.program_id(0)
    i = pl.program_id(1)

    @pl.when(p == 0)
    def _phasee():
        w0 = w0_ref[...]
        b0 = b0_ref[...]
        r = pl.ds(i * (N // PG), N // PG)
        htar_scr[r, :] = _elu(_dot(feat0_ref[...], w0) + b0)
        hmask_scr[r, :] = _elu(_dot(mask_ref[...], w0) + b0)

    @pl.when(p == 1)
    def _phase0():
        hnei0_scr[pl.ds(i * (NA // PG), NA // PG), :] = _elu(
            _dot(feat1_ref[...], w1_ref[...]) + b1_ref[...]).astype(
                jnp.bfloat16)

    @pl.when(p == 2)
    def _phase1():
        nei0 = nei0_ref[...]
        cnt0 = jnp.sum(nei0, axis=1, keepdims=True)
        cnt0 = jnp.where(cnt0 > 0, cnt0, 1.0)
        agg0 = _bdot(nei0, hnei0_scr[...]) / cnt0
        hnei1 = _elu(_dot(feat2_ref[...], w2_ref[...]) + b2_ref[...])
        nei1 = nei1_ref[...]
        cnt1 = jnp.sum(nei1, axis=1, keepdims=True)
        cnt1 = jnp.where(cnt1 > 0, cnt1, 1.0)
        agg1 = _dot(nei1, hnei1) / cnt1
        r = pl.ds(i * (N // PG), N // PG)
        h_tar = htar_scr[r, :]
        h_mask = hmask_scr[r, :]
        a0w = _dot(agg0, wagg0_ref[...])
        a1w = _dot(agg1, wagg1_ref[...])
        wg1 = wg1_ref[...]
        p_ref[:, 0 * D:1 * D] = _dot(h_tar, wg1)
        p_ref[:, 1 * D:2 * D] = _dot(_elu(h_tar + a0w), wg1)
        p_ref[:, 2 * D:3 * D] = _dot(_elu(h_mask + a0w), wg1)
        p_ref[:, 3 * D:4 * D] = _dot(_elu(h_tar + a1w), wg1)
        p_ref[:, 4 * D:5 * D] = _dot(_elu(h_mask + a1w), wg1)


def _gcn_kernel(adj0_ref, adj1_ref, madj0_ref, madj1_ref, p_ref, bg1_ref,
                wg2_ref, bg2_ref, z_ref, adj_scr, q_scr):
    i = pl.program_id(0)

    @pl.when(i < PG)
    def _stream():
        adj0 = adj0_ref[...].astype(jnp.bfloat16)
        adj1 = adj1_ref[...].astype(jnp.bfloat16)
        madj0 = madj0_ref[...].astype(jnp.bfloat16)
        madj1 = madj1_ref[...].astype(jnp.bfloat16)
        r = pl.ds(i * (N // PG), N // PG)
        adj_scr[0, r, :] = adj0
        adj_scr[1, r, :] = adj1
        adj_scr[2, r, :] = madj0
        adj_scr[3, r, :] = madj1
        pmat = p_ref[...]
        bg1 = bg1_ref[...]
        wg2 = wg2_ref[...]
        p0 = pmat[:, 0:D]
        ym = jax.nn.relu(0.5 * (_bdot(adj0, p0) + _bdot(adj1, p0)) + bg1)
        q_scr[r, 0:D] = _dot(ym, wg2)
        for v, adj in ((1, adj0), (2, madj0), (3, adj1), (4, madj1)):
            y = jax.nn.relu(_bdot(adj, pmat[:, v * D:(v + 1) * D]) + bg1)
            q_scr[r, v * D:(v + 1) * D] = _dot(y, wg2)

    @pl.when(i == PG)
    def _finish():
        q = q_scr[...].astype(jnp.bfloat16)
        bg2 = bg2_ref[...]
        q0 = q[:, 0:D]
        for t in range(PG):
            r = pl.ds(t * (N // PG), N // PG)
            a0 = adj_scr[0, r, :]
            a1 = adj_scr[1, r, :]
            z_ref[r, 0:D] = 0.5 * (_bdot(a0, q0) + _bdot(a1, q0)) + bg2
            z_ref[r, D:2 * D] = _normalize(_bdot(a0, q[:, D:2 * D]) + bg2)
            z_ref[r, 2 * D:3 * D] = _normalize(
                _bdot(adj_scr[2, r, :], q[:, 2 * D:3 * D]) + bg2)
            z_ref[r, 3 * D:4 * D] = _normalize(
                _bdot(a1, q[:, 3 * D:4 * D]) + bg2)
            z_ref[r, 4 * D:5 * D] = _normalize(
                _bdot(adj_scr[3, r, :], q[:, 4 * D:5 * D]) + bg2)


def _final_kernel(z_ref, watt_ref, batt_ref, aatt_ref, wproj_ref, bproj_ref,
                  wm1_ref, bm1_ref, wm2_ref, bm2_ref, out_ref):
    zmat = z_ref[...]
    z_coarse = zmat[:, 0:D]
    hf0 = zmat[:, D:2 * D]
    hf1 = zmat[:, 2 * D:3 * D]
    hf2 = zmat[:, 3 * D:4 * D]
    hf3 = zmat[:, 4 * D:5 * D]

    watt = watt_ref[...]
    batt = batt_ref[...]
    aatt = aatt_ref[...]

    def score(h):
        t = jnp.tanh(_dot(h, watt) + batt)
        return jnp.sum(_dot(t, aatt)) / N

    s0, s1, s2, s3 = score(hf0), score(hf1), score(hf2), score(hf3)
    m = jnp.maximum(jnp.maximum(s0, s1), jnp.maximum(s2, s3))
    e0, e1 = jnp.exp(s0 - m), jnp.exp(s1 - m)
    e2, e3 = jnp.exp(s2 - m), jnp.exp(s3 - m)
    tot = e0 + e1 + e2 + e3
    z_fine = (e0 * hf0 + e1 * hf1 + e2 * hf2 + e3 * hf3) / tot

    wproj = wproj_ref[...]
    bproj = bproj_ref[...]
    zc = _normalize(jnp.tanh(_dot(z_coarse, wproj) + bproj))
    zf = _normalize(jnp.tanh(_dot(z_fine, wproj) + bproj))

    bf16 = jnp.bfloat16
    zct = zc.T
    a = (_dot(zf, wm1_ref[...]) + bm1_ref[...]).astype(bf16)
    bt = _dot(zc, wm1_ref[...]).T.astype(bf16)
    wm2 = wm2_ref[...].astype(bf16)
    bm2 = bm2_ref[0, 0]
    T = 128
    total = jnp.float32(0.0)
    for t in range(N // T):
        r = slice(t * T, (t + 1) * T)
        zf_t = zf[r]
        e_t = jnp.exp(_dot(zf_t, zct) * (1.0 / TAU))
        a_t = a[r]
        acc = jnp.full((T, N), bm2, dtype=bf16)
        for k in range(16):
            acc = acc + jnp.tanh(a_t[:, k:k + 1] + bt[k:k + 1, :]) * wm2[:, k:k + 1]
        den_t = jnp.sum(e_t * jax.nn.sigmoid(acc.astype(jnp.float32)), axis=1)
        diag_t = jnp.sum(zf_t * zc[r], axis=1) * (1.0 / TAU)
        total = total + jnp.sum(jnp.log(den_t) - diag_t)
    out_ref[...] = jnp.reshape(total / N, (1, 1))


def kernel(feat0, feat1, feat2, mask_feat, adj0, adj1, mask_adj0, mask_adj1,
           nei0, nei1, W_fc0, b_fc0, W_fc1, b_fc1, W_fc2, b_fc2, W_agg0,
           W_agg1, W_g1, b_g1, W_g2, b_g2, W_att, b_att, a_att, W_proj,
           b_proj, W_m1, b_m1, W_m2, b_m2):
    f32 = jnp.float32
    sds = jax.ShapeDtypeStruct

    const = lambda r, c: pl.BlockSpec((r, c), lambda p, i: (0, 0))
    ph0 = lambda r, c: pl.BlockSpec(
        (r, c), lambda p, i: (jnp.where(p == 0, i, PG - 1), 0))
    ph1 = lambda r, c: pl.BlockSpec(
        (r, c), lambda p, i: (jnp.where(p == 1, i, jnp.where(p == 0, 0, PG - 1)), 0))
    ph2 = lambda r, c: pl.BlockSpec(
        (r, c), lambda p, i: (jnp.where(p == 2, i, 0), 0))

    p_mat = pl.pallas_call(
        _front_kernel,
        grid=(3, PG),
        in_specs=[
            ph0(N // PG, F0),
            ph0(N // PG, F0),
            const(F0, H),
            const(1, H),
            ph1(NA // PG, F1),
            const(F1, H),
            const(1, H),
            ph2(N // PG, NA),
            ph2(N // PG, NS),
            const(NS, F2),
            const(F2, H),
            const(1, H),
            const(H, H),
            const(H, H),
            const(H, D),
        ],
        out_specs=pl.BlockSpec((N // PG, 5 * D), lambda p, i: (i, 0)),
        out_shape=sds((N, 5 * D), f32),
        scratch_shapes=[pltpu.VMEM((N, H), f32),
                        pltpu.VMEM((N, H), f32),
                        pltpu.VMEM((NA, H), jnp.bfloat16)],
    )(feat0, mask_feat, W_fc0, b_fc0.reshape(1, H), feat1, W_fc1,
      b_fc1.reshape(1, H), nei0, nei1, feat2, W_fc2, b_fc2.reshape(1, H),
      W_agg0, W_agg1, W_g1)

    stile = lambda: pl.BlockSpec((N // PG, N),
                                 lambda i: (jnp.minimum(i, PG - 1), 0))
    cst = lambda r, c: pl.BlockSpec((r, c), lambda i: (0, 0))

    z_mat = pl.pallas_call(
        _gcn_kernel,
        grid=(PG + 1,),
        in_specs=[
            stile(),
            stile(),
            stile(),
            stile(),
            cst(N, 5 * D),
            cst(1, D),
            cst(D, D),
            cst(1, D),
        ],
        out_specs=pl.BlockSpec((N, 5 * D), lambda i: (0, 0)),
        out_shape=sds((N, 5 * D), f32),
        scratch_shapes=[pltpu.VMEM((4, N, N), jnp.bfloat16),
                        pltpu.VMEM((N, 5 * D), f32)],
    )(adj0, adj1, mask_adj0, mask_adj1, p_mat, b_g1.reshape(1, D), W_g2,
      b_g2.reshape(1, D))

    loss = pl.pallas_call(
        _final_kernel,
        out_shape=sds((1, 1), f32),
    )(z_mat, W_att, b_att.reshape(1, D), a_att.reshape(D, 1), W_proj,
      b_proj.reshape(1, D), W_m1, b_m1.reshape(1, 16), W_m2.reshape(1, 16),
      b_m2.reshape(1, 1))
    return loss[0, 0]


# speedup vs baseline: 1.1384x; 1.0039x over previous
import jax
import jax.numpy as jnp
from jax.experimental import pallas as pl
from jax.experimental.pallas import tpu as pltpu

N, NA, NS = 1024, 4096, 60
F0, F1, F2 = 1902, 334, 64
H, D = 256, 64
TAU = 0.5

EG = 4
PG = 4


def _elu(x):
    return jnp.where(x > 0, x, jnp.exp(x) - 1.0)


def _normalize(x):
    nrm = jnp.sqrt(jnp.sum(x * x, axis=1, keepdims=True))
    return x / jnp.clip(nrm, 1e-12)


def _dot(a, b):
    return jnp.dot(a, b, preferred_element_type=jnp.float32)


def _bdot(a, b):
    return jnp.dot(a.astype(jnp.bfloat16), b.astype(jnp.bfloat16),
                   preferred_element_type=jnp.float32)


def _front_kernel(feat0_ref, mask_ref, w0_ref, b0_ref, feat1_ref, w1_ref,
                  b1_ref, nei0_ref, nei1_ref, feat2_ref, w2_ref, b2_ref,
                  wagg0_ref, wagg1_ref, wg1_ref, p_ref, htar_scr, hmask_scr,
                  hnei0_scr):
    p = pl.program_id(0)
    i = pl.program_id(1)

    @pl.when(p == 0)
    def _phasee():
        w0 = w0_ref[...]
        b0 = b0_ref[...]
        r = pl.ds(i * (N // PG), N // PG)
        htar_scr[r, :] = _elu(_bdot(feat0_ref[...], w0) + b0)
        hmask_scr[r, :] = _elu(_bdot(mask_ref[...], w0) + b0)

    @pl.when(p == 1)
    def _phase0():
        hnei0_scr[pl.ds(i * (NA // PG), NA // PG), :] = _elu(
            _bdot(feat1_ref[...], w1_ref[...]) + b1_ref[...]).astype(
                jnp.bfloat16)

    @pl.when(p == 2)
    def _phase1():
        nei0 = nei0_ref[...]
        cnt0 = jnp.sum(nei0, axis=1, keepdims=True)
        cnt0 = jnp.where(cnt0 > 0, cnt0, 1.0)
        agg0 = _bdot(nei0, hnei0_scr[...]) / cnt0
        hnei1 = _elu(_dot(feat2_ref[...], w2_ref[...]) + b2_ref[...])
        nei1 = nei1_ref[...]
        cnt1 = jnp.sum(nei1, axis=1, keepdims=True)
        cnt1 = jnp.where(cnt1 > 0, cnt1, 1.0)
        agg1 = _dot(nei1, hnei1) / cnt1
        r = pl.ds(i * (N // PG), N // PG)
        h_tar = htar_scr[r, :]
        h_mask = hmask_scr[r, :]
        a0w = _dot(agg0, wagg0_ref[...])
        a1w = _dot(agg1, wagg1_ref[...])
        wg1 = wg1_ref[...]
        p_ref[:, 0 * D:1 * D] = _dot(h_tar, wg1)
        p_ref[:, 1 * D:2 * D] = _dot(_elu(h_tar + a0w), wg1)
        p_ref[:, 2 * D:3 * D] = _dot(_elu(h_mask + a0w), wg1)
        p_ref[:, 3 * D:4 * D] = _dot(_elu(h_tar + a1w), wg1)
        p_ref[:, 4 * D:5 * D] = _dot(_elu(h_mask + a1w), wg1)


def _gcn_kernel(adj0_ref, adj1_ref, madj0_ref, madj1_ref, p_ref, bg1_ref,
                wg2_ref, bg2_ref, z_ref, adj_scr, q_scr):
    i = pl.program_id(0)

    @pl.when(i < PG)
    def _stream():
        adj0 = adj0_ref[...].astype(jnp.bfloat16)
        adj1 = adj1_ref[...].astype(jnp.bfloat16)
        madj0 = madj0_ref[...].astype(jnp.bfloat16)
        madj1 = madj1_ref[...].astype(jnp.bfloat16)
        r = pl.ds(i * (N // PG), N // PG)
        adj_scr[0, r, :] = adj0
        adj_scr[1, r, :] = adj1
        adj_scr[2, r, :] = madj0
        adj_scr[3, r, :] = madj1
        pmat = p_ref[...]
        bg1 = bg1_ref[...]
        wg2 = wg2_ref[...]
        p0 = pmat[:, 0:D]
        ym = jax.nn.relu(0.5 * (_bdot(adj0, p0) + _bdot(adj1, p0)) + bg1)
        q_scr[r, 0:D] = _dot(ym, wg2)
        for v, adj in ((1, adj0), (2, madj0), (3, adj1), (4, madj1)):
            y = jax.nn.relu(_bdot(adj, pmat[:, v * D:(v + 1) * D]) + bg1)
            q_scr[r, v * D:(v + 1) * D] = _dot(y, wg2)

    @pl.when(i == PG)
    def _finish():
        q = q_scr[...].astype(jnp.bfloat16)
        bg2 = bg2_ref[...]
        q0 = q[:, 0:D]
        for t in range(PG):
            r = pl.ds(t * (N // PG), N // PG)
            a0 = adj_scr[0, r, :]
            a1 = adj_scr[1, r, :]
            z_ref[r, 0:D] = 0.5 * (_bdot(a0, q0) + _bdot(a1, q0)) + bg2
            z_ref[r, D:2 * D] = _normalize(_bdot(a0, q[:, D:2 * D]) + bg2)
            z_ref[r, 2 * D:3 * D] = _normalize(
                _bdot(adj_scr[2, r, :], q[:, 2 * D:3 * D]) + bg2)
            z_ref[r, 3 * D:4 * D] = _normalize(
                _bdot(a1, q[:, 3 * D:4 * D]) + bg2)
            z_ref[r, 4 * D:5 * D] = _normalize(
                _bdot(adj_scr[3, r, :], q[:, 4 * D:5 * D]) + bg2)


def _final_kernel(z_ref, watt_ref, batt_ref, aatt_ref, wproj_ref, bproj_ref,
                  wm1_ref, bm1_ref, wm2_ref, bm2_ref, out_ref):
    zmat = z_ref[...]
    z_coarse = zmat[:, 0:D]
    hf0 = zmat[:, D:2 * D]
    hf1 = zmat[:, 2 * D:3 * D]
    hf2 = zmat[:, 3 * D:4 * D]
    hf3 = zmat[:, 4 * D:5 * D]

    watt = watt_ref[...]
    batt = batt_ref[...]
    aatt = aatt_ref[...]

    def score(h):
        t = jnp.tanh(_dot(h, watt) + batt)
        return jnp.sum(_dot(t, aatt)) / N

    s0, s1, s2, s3 = score(hf0), score(hf1), score(hf2), score(hf3)
    m = jnp.maximum(jnp.maximum(s0, s1), jnp.maximum(s2, s3))
    e0, e1 = jnp.exp(s0 - m), jnp.exp(s1 - m)
    e2, e3 = jnp.exp(s2 - m), jnp.exp(s3 - m)
    tot = e0 + e1 + e2 + e3
    z_fine = (e0 * hf0 + e1 * hf1 + e2 * hf2 + e3 * hf3) / tot

    wproj = wproj_ref[...]
    bproj = bproj_ref[...]
    zc = _normalize(jnp.tanh(_dot(z_coarse, wproj) + bproj))
    zf = _normalize(jnp.tanh(_dot(z_fine, wproj) + bproj))

    bf16 = jnp.bfloat16
    zct = zc.T
    a = (_dot(zf, wm1_ref[...]) + bm1_ref[...]).astype(bf16)
    bt = _dot(zc, wm1_ref[...]).T.astype(bf16)
    wm2 = wm2_ref[...].astype(bf16)
    bm2 = bm2_ref[0, 0]
    T = 128
    total = jnp.float32(0.0)
    for t in range(N // T):
        r = slice(t * T, (t + 1) * T)
        zf_t = zf[r]
        e_t = jnp.exp(_dot(zf_t, zct) * (1.0 / TAU))
        a_t = a[r]
        acc = jnp.full((T, N), bm2, dtype=bf16)
        for k in range(16):
            acc = acc + jnp.tanh(a_t[:, k:k + 1] + bt[k:k + 1, :]) * wm2[:, k:k + 1]
        den_t = jnp.sum(e_t * jax.nn.sigmoid(acc.astype(jnp.float32)), axis=1)
        diag_t = jnp.sum(zf_t * zc[r], axis=1) * (1.0 / TAU)
        total = total + jnp.sum(jnp.log(den_t) - diag_t)
    out_ref[...] = jnp.reshape(total / N, (1, 1))


def kernel(feat0, feat1, feat2, mask_feat, adj0, adj1, mask_adj0, mask_adj1,
           nei0, nei1, W_fc0, b_fc0, W_fc1, b_fc1, W_fc2, b_fc2, W_agg0,
           W_agg1, W_g1, b_g1, W_g2, b_g2, W_att, b_att, a_att, W_proj,
           b_proj, W_m1, b_m1, W_m2, b_m2):
    f32 = jnp.float32
    sds = jax.ShapeDtypeStruct

    const = lambda r, c: pl.BlockSpec((r, c), lambda p, i: (0, 0))
    ph0 = lambda r, c: pl.BlockSpec(
        (r, c), lambda p, i: (jnp.where(p == 0, i, PG - 1), 0))
    ph1 = lambda r, c: pl.BlockSpec(
        (r, c), lambda p, i: (jnp.where(p == 1, i, jnp.where(p == 0, 0, PG - 1)), 0))
    ph2 = lambda r, c: pl.BlockSpec(
        (r, c), lambda p, i: (jnp.where(p == 2, i, 0), 0))

    p_mat = pl.pallas_call(
        _front_kernel,
        grid=(3, PG),
        in_specs=[
            ph0(N // PG, F0),
            ph0(N // PG, F0),
            const(F0, H),
            const(1, H),
            ph1(NA // PG, F1),
            const(F1, H),
            const(1, H),
            ph2(N // PG, NA),
            ph2(N // PG, NS),
            const(NS, F2),
            const(F2, H),
            const(1, H),
            const(H, H),
            const(H, H),
            const(H, D),
        ],
        out_specs=pl.BlockSpec((N // PG, 5 * D), lambda p, i: (i, 0)),
        out_shape=sds((N, 5 * D), f32),
        scratch_shapes=[pltpu.VMEM((N, H), f32),
                        pltpu.VMEM((N, H), f32),
                        pltpu.VMEM((NA, H), jnp.bfloat16)],
    )(feat0, mask_feat, W_fc0, b_fc0.reshape(1, H), feat1, W_fc1,
      b_fc1.reshape(1, H), nei0, nei1, feat2, W_fc2, b_fc2.reshape(1, H),
      W_agg0, W_agg1, W_g1)

    stile = lambda: pl.BlockSpec((N // PG, N),
                                 lambda i: (jnp.minimum(i, PG - 1), 0))
    cst = lambda r, c: pl.BlockSpec((r, c), lambda i: (0, 0))

    z_mat = pl.pallas_call(
        _gcn_kernel,
        grid=(PG + 1,),
        in_specs=[
            stile(),
            stile(),
            stile(),
            stile(),
            cst(N, 5 * D),
            cst(1, D),
            cst(D, D),
            cst(1, D),
        ],
        out_specs=pl.BlockSpec((N, 5 * D), lambda i: (0, 0)),
        out_shape=sds((N, 5 * D), f32),
        scratch_shapes=[pltpu.VMEM((4, N, N), jnp.bfloat16),
                        pltpu.VMEM((N, 5 * D), f32)],
    )(adj0, adj1, mask_adj0, mask_adj1, p_mat, b_g1.reshape(1, D), W_g2,
      b_g2.reshape(1, D))

    loss = pl.pallas_call(
        _final_kernel,
        out_shape=sds((1, 1), f32),
    )(z_mat, W_att, b_att.reshape(1, D), a_att.reshape(D, 1), W_proj,
      b_proj.reshape(1, D), W_m1, b_m1.reshape(1, 16), W_m2.reshape(1, 16),
      b_m2.reshape(1, 1))
    return loss[0, 0]


# speedup vs baseline: 1.1489x; 1.0093x over previous
import jax
import jax.numpy as jnp
from jax.experimental import pallas as pl
from jax.experimental.pallas import tpu as pltpu

N, NA, NS = 1024, 4096, 60
F0, F1, F2 = 1902, 334, 64
H, D = 256, 64
TAU = 0.5

EG = 4
PG = 4


def _elu(x):
    return jnp.where(x > 0, x, jnp.exp(x) - 1.0)


def _normalize(x):
    nrm = jnp.sqrt(jnp.sum(x * x, axis=1, keepdims=True))
    return x / jnp.clip(nrm, 1e-12)


def _dot(a, b):
    return jnp.dot(a, b, preferred_element_type=jnp.float32)


def _bdot(a, b):
    return jnp.dot(a.astype(jnp.bfloat16), b.astype(jnp.bfloat16),
                   preferred_element_type=jnp.float32)


def _front_kernel(feat0_ref, mask_ref, w0_ref, b0_ref, feat1_ref, w1_ref,
                  b1_ref, nei0_ref, nei1_ref, feat2_ref, w2_ref, b2_ref,
                  wagg0_ref, wagg1_ref, wg1_ref, p_ref, htar_scr, hmask_scr,
                  hnei0_scr):
    p = pl.program_id(0)
    i = pl.program_id(1)

    @pl.when(p == 0)
    def _phasee():
        w0 = w0_ref[...]
        b0 = b0_ref[...]
        r = pl.ds(i * (N // PG), N // PG)
        htar_scr[r, :] = _elu(_bdot(feat0_ref[...], w0) + b0)
        hmask_scr[r, :] = _elu(_bdot(mask_ref[...], w0) + b0)

    @pl.when(p == 1)
    def _phase0():
        hnei0_scr[pl.ds(i * (NA // PG), NA // PG), :] = _elu(
            _bdot(feat1_ref[...], w1_ref[...]) + b1_ref[...]).astype(
                jnp.bfloat16)

    @pl.when(p == 2)
    def _phase1():
        nei0 = nei0_ref[...]
        cnt0 = jnp.sum(nei0, axis=1, keepdims=True)
        cnt0 = jnp.where(cnt0 > 0, cnt0, 1.0)
        agg0 = _bdot(nei0, hnei0_scr[...]) / cnt0
        hnei1 = _elu(_dot(feat2_ref[...], w2_ref[...]) + b2_ref[...])
        nei1 = nei1_ref[...]
        cnt1 = jnp.sum(nei1, axis=1, keepdims=True)
        cnt1 = jnp.where(cnt1 > 0, cnt1, 1.0)
        agg1 = _dot(nei1, hnei1) / cnt1
        r = pl.ds(i * (N // PG), N // PG)
        h_tar = htar_scr[r, :]
        h_mask = hmask_scr[r, :]
        a0w = _dot(agg0, wagg0_ref[...])
        a1w = _dot(agg1, wagg1_ref[...])
        wg1 = wg1_ref[...]
        p_ref[:, 0 * D:1 * D] = _dot(h_tar, wg1)
        p_ref[:, 1 * D:2 * D] = _dot(_elu(h_tar + a0w), wg1)
        p_ref[:, 2 * D:3 * D] = _dot(_elu(h_mask + a0w), wg1)
        p_ref[:, 3 * D:4 * D] = _dot(_elu(h_tar + a1w), wg1)
        p_ref[:, 4 * D:5 * D] = _dot(_elu(h_mask + a1w), wg1)


def _gcn_kernel(adj0_ref, adj1_ref, madj0_ref, madj1_ref, p_ref, bg1_ref,
                wg2_ref, bg2_ref, z_ref, adj_scr, q_scr):
    i = pl.program_id(0)

    @pl.when(i < PG)
    def _stream():
        adj0 = adj0_ref[...].astype(jnp.bfloat16)
        adj1 = adj1_ref[...].astype(jnp.bfloat16)
        madj0 = madj0_ref[...].astype(jnp.bfloat16)
        madj1 = madj1_ref[...].astype(jnp.bfloat16)
        r = pl.ds(i * (N // PG), N // PG)
        adj_scr[0, r, :] = adj0
        adj_scr[1, r, :] = adj1
        adj_scr[2, r, :] = madj0
        adj_scr[3, r, :] = madj1
        pmat = p_ref[...]
        bg1 = bg1_ref[...]
        wg2 = wg2_ref[...]
        p0 = pmat[:, 0:D]
        ym = jax.nn.relu(0.5 * (_bdot(adj0, p0) + _bdot(adj1, p0)) + bg1)
        q_scr[r, 0:D] = _dot(ym, wg2)
        for v, adj in ((1, adj0), (2, madj0), (3, adj1), (4, madj1)):
            y = jax.nn.relu(_bdot(adj, pmat[:, v * D:(v + 1) * D]) + bg1)
            q_scr[r, v * D:(v + 1) * D] = _dot(y, wg2)

    @pl.when(i == PG)
    def _finish():
        q = q_scr[...].astype(jnp.bfloat16)
        bg2 = bg2_ref[...]
        q0 = q[:, 0:D]
        for t in range(PG):
            r = pl.ds(t * (N // PG), N // PG)
            a0 = adj_scr[0, r, :]
            a1 = adj_scr[1, r, :]
            z_ref[r, 0:D] = 0.5 * (_bdot(a0, q0) + _bdot(a1, q0)) + bg2
            z_ref[r, D:2 * D] = _normalize(_bdot(a0, q[:, D:2 * D]) + bg2)
            z_ref[r, 2 * D:3 * D] = _normalize(
                _bdot(adj_scr[2, r, :], q[:, 2 * D:3 * D]) + bg2)
            z_ref[r, 3 * D:4 * D] = _normalize(
                _bdot(a1, q[:, 3 * D:4 * D]) + bg2)
            z_ref[r, 4 * D:5 * D] = _normalize(
                _bdot(adj_scr[3, r, :], q[:, 4 * D:5 * D]) + bg2)


def _final_kernel(z_ref, watt_ref, batt_ref, aatt_ref, wproj_ref, bproj_ref,
                  wm1_ref, bm1_ref, wm2_ref, bm2_ref, out_ref):
    zmat = z_ref[...]
    z_coarse = zmat[:, 0:D]
    hf0 = zmat[:, D:2 * D]
    hf1 = zmat[:, 2 * D:3 * D]
    hf2 = zmat[:, 3 * D:4 * D]
    hf3 = zmat[:, 4 * D:5 * D]

    watt = watt_ref[...]
    batt = batt_ref[...]
    aatt = aatt_ref[...]

    def score(h):
        t = jnp.tanh(_dot(h, watt) + batt)
        return jnp.sum(_dot(t, aatt)) / N

    s0, s1, s2, s3 = score(hf0), score(hf1), score(hf2), score(hf3)
    m = jnp.maximum(jnp.maximum(s0, s1), jnp.maximum(s2, s3))
    e0, e1 = jnp.exp(s0 - m), jnp.exp(s1 - m)
    e2, e3 = jnp.exp(s2 - m), jnp.exp(s3 - m)
    tot = e0 + e1 + e2 + e3
    z_fine = (e0 * hf0 + e1 * hf1 + e2 * hf2 + e3 * hf3) / tot

    wproj = wproj_ref[...]
    bproj = bproj_ref[...]
    zc = _normalize(jnp.tanh(_dot(z_coarse, wproj) + bproj))
    zf = _normalize(jnp.tanh(_dot(z_fine, wproj) + bproj))

    bf16 = jnp.bfloat16
    zct = zc.T
    a = (_dot(zf, wm1_ref[...]) + bm1_ref[...]).astype(bf16)
    bt = _dot(zc, wm1_ref[...]).T.astype(bf16)
    wm2 = wm2_ref[...].astype(bf16)
    bm2 = bm2_ref[0, 0]
    T = 256
    total = jnp.float32(0.0)
    for t in range(N // T):
        r = slice(t * T, (t + 1) * T)
        zf_t = zf[r]
        e_t = jnp.exp(_dot(zf_t, zct) * (1.0 / TAU))
        a_t = a[r]
        acc = jnp.full((T, N), bm2, dtype=bf16)
        for k in range(16):
            acc = acc + jnp.tanh(a_t[:, k:k + 1] + bt[k:k + 1, :]) * wm2[:, k:k + 1]
        den_t = jnp.sum(e_t * jax.nn.sigmoid(acc.astype(jnp.float32)), axis=1)
        diag_t = jnp.sum(zf_t * zc[r], axis=1) * (1.0 / TAU)
        total = total + jnp.sum(jnp.log(den_t) - diag_t)
    out_ref[...] = jnp.reshape(total / N, (1, 1))


def kernel(feat0, feat1, feat2, mask_feat, adj0, adj1, mask_adj0, mask_adj1,
           nei0, nei1, W_fc0, b_fc0, W_fc1, b_fc1, W_fc2, b_fc2, W_agg0,
           W_agg1, W_g1, b_g1, W_g2, b_g2, W_att, b_att, a_att, W_proj,
           b_proj, W_m1, b_m1, W_m2, b_m2):
    f32 = jnp.float32
    sds = jax.ShapeDtypeStruct

    const = lambda r, c: pl.BlockSpec((r, c), lambda p, i: (0, 0))
    ph0 = lambda r, c: pl.BlockSpec(
        (r, c), lambda p, i: (jnp.where(p == 0, i, PG - 1), 0))
    ph1 = lambda r, c: pl.BlockSpec(
        (r, c), lambda p, i: (jnp.where(p == 1, i, jnp.where(p == 0, 0, PG - 1)), 0))
    ph2 = lambda r, c: pl.BlockSpec(
        (r, c), lambda p, i: (jnp.where(p == 2, i, 0), 0))

    p_mat = pl.pallas_call(
        _front_kernel,
        grid=(3, PG),
        in_specs=[
            ph0(N // PG, F0),
            ph0(N // PG, F0),
            const(F0, H),
            const(1, H),
            ph1(NA // PG, F1),
            const(F1, H),
            const(1, H),
            ph2(N // PG, NA),
            ph2(N // PG, NS),
            const(NS, F2),
            const(F2, H),
            const(1, H),
            const(H, H),
            const(H, H),
            const(H, D),
        ],
        out_specs=pl.BlockSpec((N // PG, 5 * D), lambda p, i: (i, 0)),
        out_shape=sds((N, 5 * D), f32),
        scratch_shapes=[pltpu.VMEM((N, H), f32),
                        pltpu.VMEM((N, H), f32),
                        pltpu.VMEM((NA, H), jnp.bfloat16)],
    )(feat0, mask_feat, W_fc0, b_fc0.reshape(1, H), feat1, W_fc1,
      b_fc1.reshape(1, H), nei0, nei1, feat2, W_fc2, b_fc2.reshape(1, H),
      W_agg0, W_agg1, W_g1)

    stile = lambda: pl.BlockSpec((N // PG, N),
                                 lambda i: (jnp.minimum(i, PG - 1), 0))
    cst = lambda r, c: pl.BlockSpec((r, c), lambda i: (0, 0))

    z_mat = pl.pallas_call(
        _gcn_kernel,
        grid=(PG + 1,),
        in_specs=[
            stile(),
            stile(),
            stile(),
            stile(),
            cst(N, 5 * D),
            cst(1, D),
            cst(D, D),
            cst(1, D),
        ],
        out_specs=pl.BlockSpec((N, 5 * D), lambda i: (0, 0)),
        out_shape=sds((N, 5 * D), f32),
        scratch_shapes=[pltpu.VMEM((4, N, N), jnp.bfloat16),
                        pltpu.VMEM((N, 5 * D), f32)],
    )(adj0, adj1, mask_adj0, mask_adj1, p_mat, b_g1.reshape(1, D), W_g2,
      b_g2.reshape(1, D))

    loss = pl.pallas_call(
        _final_kernel,
        out_shape=sds((1, 1), f32),
    )(z_mat, W_att, b_att.reshape(1, D), a_att.reshape(D, 1), W_proj,
      b_proj.reshape(1, D), W_m1, b_m1.reshape(1, 16), W_m2.reshape(1, 16),
      b_m2.reshape(1, 1))
    return loss[0, 0]
